# baseline (speedup 1.0000x reference)
import functools

import jax
import jax.numpy as jnp
from jax import lax
from jax.experimental import pallas as pl
from jax.experimental.pallas import tpu as pltpu

PB = 512


def _vq_body(w_ref, z_ref, cb_ref, zq_ref, loss_ref, *, size, inv_bs):
    b = pl.program_id(0)
    p = pl.program_id(1)
    w = w_ref[0, 0]
    zb = z_ref[0]
    cb = cb_ref[...]

    cbw = cb * (2.0 * w)
    cbsqw = w * jnp.sum(cb * cb, axis=1)
    g = lax.dot_general(zb, cbw, (((0,), (1,)), ((), ())),
                        preferred_element_type=jnp.float32) - cbsqw[None, :]

    rowmax = jnp.max(g, axis=1)
    t = g - rowmax[:, None]
    e = jnp.exp(t)
    et = e * t
    onehot = (t == 0.0).astype(jnp.float32)

    ones = jnp.ones((size, 1), jnp.float32)
    dn = (((1,), (0,)), ((), ()))
    denom = lax.dot_general(e, ones, dn, preferred_element_type=jnp.float32)
    num = lax.dot_general(et, ones, dn, preferred_element_type=jnp.float32)
    cnt = lax.dot_general(onehot, ones, dn, preferred_element_type=jnp.float32)
    kld = jnp.sum(num / denom - jnp.log(denom))

    zq_ref[0] = lax.dot_general(cb, onehot, (((0,), (1,)), ((), ())),
                                preferred_element_type=jnp.float32)

    @pl.when(jnp.max(cnt) > 1.5)
    def _fix_ties():
        iota = lax.broadcasted_iota(jnp.int32, (PB, size), 1)
        idx = jnp.min(jnp.where(t == 0.0, iota, size), axis=1)
        oh1 = (iota == idx[:, None]).astype(jnp.float32)
        zq_ref[0] = lax.dot_general(cb, oh1, (((0,), (1,)), ((), ())),
                                    preferred_element_type=jnp.float32)

    zqf = zq_ref[0]
    commit = w * jnp.sum((zb - zqf) ** 2)

    @pl.when((b == 0) & (p == 0))
    def _init():
        loss_ref[0, 0] = 0.0

    loss_ref[0, 0] += (kld + commit) * inv_bs


def kernel(z, codebook, var_q, var_init):
    bs, dim_z, d1, d2 = z.shape
    size, _ = codebook.shape
    npix = d1 * d2
    z3 = z.reshape(bs, dim_z, npix)

    var_q_eff = jax.nn.sigmoid(var_q) * 2.0 * var_init
    w = (0.5 / jnp.clip(var_q_eff, 1e-10, None)).reshape(1, 1)

    body = functools.partial(_vq_body, size=size, inv_bs=1.0 / bs)
    zq3, loss = pl.pallas_call(
        body,
        grid=(bs, npix // PB),
        in_specs=[
            pl.BlockSpec(memory_space=pltpu.SMEM),
            pl.BlockSpec((1, dim_z, PB), lambda b, p: (b, 0, p)),
            pl.BlockSpec((size, dim_z), lambda b, p: (0, 0)),
        ],
        out_specs=[
            pl.BlockSpec((1, dim_z, PB), lambda b, p: (b, 0, p)),
            pl.BlockSpec(memory_space=pltpu.SMEM),
        ],
        out_shape=[
            jax.ShapeDtypeStruct((bs, dim_z, npix), jnp.float32),
            jax.ShapeDtypeStruct((1, 1), jnp.float32),
        ],
    )(w, z3, codebook)
    return zq3.reshape(bs, dim_z, d1, d2), loss[0, 0]


# speedup vs baseline: 1.0386x; 1.0386x over previous
import functools

import jax
import jax.numpy as jnp
from jax import lax
from jax.experimental import pallas as pl
from jax.experimental.pallas import tpu as pltpu

PB = 512


def _vq_body(w_ref, z_ref, cb_ref, zq_ref, loss_ref, cbw_s, nb_s, *, size,
             inv_bs):
    b = pl.program_id(0)
    p = pl.program_id(1)
    w = w_ref[0, 0]
    zb = z_ref[0]
    cb = cb_ref[...]

    @pl.when((b == 0) & (p == 0))
    def _prep():
        cbw_s[...] = cb * (2.0 * w)
        nb_s[...] = (-w) * jnp.sum(cb * cb, axis=1)[None, :]

    g = lax.dot_general(zb, cbw_s[...], (((0,), (1,)), ((), ())),
                        preferred_element_type=jnp.float32) + nb_s[...]

    rowmax = jnp.max(g, axis=1)
    t = g - rowmax[:, None]
    e = jnp.exp(t)
    et = e * t
    denom = jnp.sum(e, axis=1)
    num = jnp.sum(et, axis=1)
    kld = jnp.sum(num / denom - jnp.log(denom))

    onehot = (t == 0.0).astype(jnp.float32)
    zq_ref[0] = lax.dot_general(cb, onehot, (((0,), (1,)), ((), ())),
                                preferred_element_type=jnp.float32)

    @pl.when(jnp.sum(onehot) != float(PB))
    def _fix_ties():
        iota = lax.broadcasted_iota(jnp.int32, (PB, size), 1)
        idx = jnp.min(jnp.where(t == 0.0, iota, size), axis=1)
        oh1 = (iota == idx[:, None]).astype(jnp.float32)
        zq_ref[0] = lax.dot_general(cb, oh1, (((0,), (1,)), ((), ())),
                                    preferred_element_type=jnp.float32)

    commit = w * jnp.sum(zb * zb) - jnp.sum(rowmax)

    @pl.when((b == 0) & (p == 0))
    def _init():
        loss_ref[0, 0] = 0.0

    loss_ref[0, 0] += (kld + commit) * inv_bs


def kernel(z, codebook, var_q, var_init):
    bs, dim_z, d1, d2 = z.shape
    size, _ = codebook.shape
    npix = d1 * d2
    z3 = z.reshape(bs, dim_z, npix)

    var_q_eff = jax.nn.sigmoid(var_q) * 2.0 * var_init
    w = (0.5 / jnp.clip(var_q_eff, 1e-10, None)).reshape(1, 1)

    body = functools.partial(_vq_body, size=size, inv_bs=1.0 / bs)
    zq3, loss = pl.pallas_call(
        body,
        grid=(bs, npix // PB),
        in_specs=[
            pl.BlockSpec(memory_space=pltpu.SMEM),
            pl.BlockSpec((1, dim_z, PB), lambda b, p: (b, 0, p)),
            pl.BlockSpec((size, dim_z), lambda b, p: (0, 0)),
        ],
        out_specs=[
            pl.BlockSpec((1, dim_z, PB), lambda b, p: (b, 0, p)),
            pl.BlockSpec(memory_space=pltpu.SMEM),
        ],
        out_shape=[
            jax.ShapeDtypeStruct((bs, dim_z, npix), jnp.float32),
            jax.ShapeDtypeStruct((1, 1), jnp.float32),
        ],
        scratch_shapes=[
            pltpu.VMEM((size, dim_z), jnp.float32),
            pltpu.VMEM((1, size), jnp.float32),
        ],
    )(w, z3, codebook)
    return zq3.reshape(bs, dim_z, d1, d2), loss[0, 0]


# speedup vs baseline: 1.1487x; 1.1060x over previous
import functools

import jax
import jax.numpy as jnp
from jax import lax
from jax.experimental import pallas as pl
from jax.experimental.pallas import tpu as pltpu

PB = 512


def _vq_body(w_ref, z_ref, cb_ref, zq_ref, loss_ref, cbw_s, nb_s, rev_s, *,
             size, inv_bs):
    b = pl.program_id(0)
    p = pl.program_id(1)
    w = w_ref[0, 0]
    zb = z_ref[0]
    cb = cb_ref[...]

    @pl.when((b == 0) & (p == 0))
    def _prep():
        cbw_s[...] = cb * (2.0 * w)
        nb_s[...] = (-w) * jnp.sum(cb * cb, axis=1)[None, :]
        rev_s[...] = (jnp.int32(size) - lax.broadcasted_iota(
            jnp.int32, (1, size), 1)).astype(jnp.float32)

    g = lax.dot_general(zb, cbw_s[...], (((0,), (1,)), ((), ())),
                        preferred_element_type=jnp.float32) + nb_s[...]

    rowmax = jnp.max(g, axis=1)
    t = g - rowmax[:, None]
    e = jnp.exp(t)
    et = e * t
    denom = jnp.sum(e, axis=1)
    num = jnp.sum(et, axis=1)
    kld = jnp.sum(num / denom - jnp.log(denom))

    val = jnp.where(t == 0.0, rev_s[...], 0.0)
    vmax = jnp.max(val, axis=1)
    onehot = (val == vmax[:, None]).astype(jnp.float32)
    zq_ref[0] = lax.dot_general(cb, onehot, (((0,), (1,)), ((), ())),
                                preferred_element_type=jnp.float32)

    commit = w * jnp.sum(zb * zb) - jnp.sum(rowmax)

    @pl.when((b == 0) & (p == 0))
    def _init():
        loss_ref[0, 0] = 0.0

    loss_ref[0, 0] += (kld + commit) * inv_bs


def kernel(z, codebook, var_q, var_init):
    bs, dim_z, d1, d2 = z.shape
    size, _ = codebook.shape
    npix = d1 * d2
    z3 = z.reshape(bs, dim_z, npix)

    var_q_eff = jax.nn.sigmoid(var_q) * 2.0 * var_init
    w = (0.5 / jnp.clip(var_q_eff, 1e-10, None)).reshape(1, 1)

    body = functools.partial(_vq_body, size=size, inv_bs=1.0 / bs)
    zq3, loss = pl.pallas_call(
        body,
        grid=(bs, npix // PB),
        in_specs=[
            pl.BlockSpec(memory_space=pltpu.SMEM),
            pl.BlockSpec((1, dim_z, PB), lambda b, p: (b, 0, p)),
            pl.BlockSpec((size, dim_z), lambda b, p: (0, 0)),
        ],
        out_specs=[
            pl.BlockSpec((1, dim_z, PB), lambda b, p: (b, 0, p)),
            pl.BlockSpec(memory_space=pltpu.SMEM),
        ],
        out_shape=[
            jax.ShapeDtypeStruct((bs, dim_z, npix), jnp.float32),
            jax.ShapeDtypeStruct((1, 1), jnp.float32),
        ],
        scratch_shapes=[
            pltpu.VMEM((size, dim_z), jnp.float32),
            pltpu.VMEM((1, size), jnp.float32),
            pltpu.VMEM((1, size), jnp.float32),
        ],
    )(w, z3, codebook)
    return zq3.reshape(bs, dim_z, d1, d2), loss[0, 0]


# speedup vs baseline: 1.2633x; 1.0998x over previous
import functools

import jax
import jax.numpy as jnp
from jax import lax
from jax.experimental import pallas as pl
from jax.experimental.pallas import tpu as pltpu

PB = 1024


def _vq_body(w_ref, z_ref, cb_ref, zq_ref, loss_ref, cbw_s, nb_s, rev_s, *,
             size, inv_bs):
    b = pl.program_id(0)
    p = pl.program_id(1)
    w = w_ref[0, 0]
    zb = z_ref[0]
    cb = cb_ref[...]

    @pl.when((b == 0) & (p == 0))
    def _prep():
        cbw_s[...] = cb * (2.0 * w)
        nb_s[...] = (-w) * jnp.sum(cb * cb, axis=1)[None, :]
        rev_s[...] = (jnp.int32(size) - lax.broadcasted_iota(
            jnp.int32, (1, size), 1)).astype(jnp.float32)

    g = lax.dot_general(zb, cbw_s[...], (((0,), (1,)), ((), ())),
                        preferred_element_type=jnp.float32) + nb_s[...]

    rowmax = jnp.max(g, axis=1)
    t = g - rowmax[:, None]
    e = jnp.exp(t)
    et = e * t
    denom = jnp.sum(e, axis=1)
    num = jnp.sum(et, axis=1)
    kld = jnp.sum(num / denom - jnp.log(denom))

    val = jnp.where(t == 0.0, rev_s[...], 0.0)
    vmax = jnp.max(val, axis=1)
    onehot = (val == vmax[:, None]).astype(jnp.float32)
    zq_ref[0] = lax.dot_general(cb, onehot, (((0,), (1,)), ((), ())),
                                preferred_element_type=jnp.float32)

    commit = w * jnp.sum(zb * zb) - jnp.sum(rowmax)

    @pl.when((b == 0) & (p == 0))
    def _init():
        loss_ref[0, 0] = 0.0

    loss_ref[0, 0] += (kld + commit) * inv_bs


def kernel(z, codebook, var_q, var_init):
    bs, dim_z, d1, d2 = z.shape
    size, _ = codebook.shape
    npix = d1 * d2
    z3 = z.reshape(bs, dim_z, npix)

    var_q_eff = jax.nn.sigmoid(var_q) * 2.0 * var_init
    w = (0.5 / jnp.clip(var_q_eff, 1e-10, None)).reshape(1, 1)

    body = functools.partial(_vq_body, size=size, inv_bs=1.0 / bs)
    zq3, loss = pl.pallas_call(
        body,
        grid=(bs, npix // PB),
        in_specs=[
            pl.BlockSpec(memory_space=pltpu.SMEM),
            pl.BlockSpec((1, dim_z, PB), lambda b, p: (b, 0, p)),
            pl.BlockSpec((size, dim_z), lambda b, p: (0, 0)),
        ],
        out_specs=[
            pl.BlockSpec((1, dim_z, PB), lambda b, p: (b, 0, p)),
            pl.BlockSpec(memory_space=pltpu.SMEM),
        ],
        out_shape=[
            jax.ShapeDtypeStruct((bs, dim_z, npix), jnp.float32),
            jax.ShapeDtypeStruct((1, 1), jnp.float32),
        ],
        scratch_shapes=[
            pltpu.VMEM((size, dim_z), jnp.float32),
            pltpu.VMEM((1, size), jnp.float32),
            pltpu.VMEM((1, size), jnp.float32),
        ],
    )(w, z3, codebook)
    return zq3.reshape(bs, dim_z, d1, d2), loss[0, 0]


# speedup vs baseline: 1.3822x; 1.0941x over previous
import functools

import jax
import jax.numpy as jnp
from jax import lax
from jax.experimental import pallas as pl
from jax.experimental.pallas import tpu as pltpu

PB = 1024


def _vq_body(w_ref, z_ref, cb_ref, zq_ref, loss_ref, cbw_s, nb_s, rev_s, *,
             size, inv_bs):
    b = pl.program_id(0)
    p = pl.program_id(1)
    w = w_ref[0, 0]
    zb = z_ref[0]
    cb = cb_ref[...]

    @pl.when((b == 0) & (p == 0))
    def _prep():
        cbw_s[...] = cb * (2.0 * w)
        nb_s[...] = (-w) * jnp.sum(cb * cb, axis=1)[None, :]
        rev_s[...] = (jnp.int32(size) - lax.broadcasted_iota(
            jnp.int32, (1, size), 1)).astype(jnp.float32)

    g = lax.dot_general(zb, cbw_s[...], (((0,), (1,)), ((), ())),
                        preferred_element_type=jnp.float32) + nb_s[...]

    rowmax = jnp.max(g, axis=1)
    e = jnp.exp(g - rowmax[:, None])
    denom = jnp.sum(e, axis=1)
    sumeg = jnp.sum(e * g, axis=1)
    kld = jnp.sum(sumeg / denom - rowmax - jnp.log(denom))

    rev = rev_s[...]
    vmax = jnp.max(jnp.where(g == rowmax[:, None], rev, 0.0), axis=1)
    onehot = (rev == vmax[:, None]).astype(jnp.float32)
    zq_ref[0] = lax.dot_general(cb, onehot, (((0,), (1,)), ((), ())),
                                preferred_element_type=jnp.float32)

    commit = w * jnp.sum(zb * zb) - jnp.sum(rowmax)

    @pl.when((b == 0) & (p == 0))
    def _init():
        loss_ref[0, 0] = 0.0

    loss_ref[0, 0] += (kld + commit) * inv_bs


def kernel(z, codebook, var_q, var_init):
    bs, dim_z, d1, d2 = z.shape
    size, _ = codebook.shape
    npix = d1 * d2
    z3 = z.reshape(bs, dim_z, npix)

    var_q_eff = jax.nn.sigmoid(var_q) * 2.0 * var_init
    w = (0.5 / jnp.clip(var_q_eff, 1e-10, None)).reshape(1, 1)

    body = functools.partial(_vq_body, size=size, inv_bs=1.0 / bs)
    zq3, loss = pl.pallas_call(
        body,
        grid=(bs, npix // PB),
        in_specs=[
            pl.BlockSpec(memory_space=pltpu.SMEM),
            pl.BlockSpec((1, dim_z, PB), lambda b, p: (b, 0, p)),
            pl.BlockSpec((size, dim_z), lambda b, p: (0, 0)),
        ],
        out_specs=[
            pl.BlockSpec((1, dim_z, PB), lambda b, p: (b, 0, p)),
            pl.BlockSpec(memory_space=pltpu.SMEM),
        ],
        out_shape=[
            jax.ShapeDtypeStruct((bs, dim_z, npix), jnp.float32),
            jax.ShapeDtypeStruct((1, 1), jnp.float32),
        ],
        scratch_shapes=[
            pltpu.VMEM((size, dim_z), jnp.float32),
            pltpu.VMEM((1, size), jnp.float32),
            pltpu.VMEM((1, size), jnp.float32),
        ],
    )(w, z3, codebook)
    return zq3.reshape(bs, dim_z, d1, d2), loss[0, 0]


# speedup vs baseline: 1.4353x; 1.0384x over previous
import functools

import jax
import jax.numpy as jnp
from jax import lax
from jax.experimental import pallas as pl
from jax.experimental.pallas import tpu as pltpu

NB2 = 2


def _vq_body(w_ref, z_ref, cb_ref, zq_ref, loss_ref, cbw_s, nb_s, rev_s, *,
             size, inv_bs):
    step = pl.program_id(0)
    w = w_ref[0, 0]
    cb = cb_ref[...]

    @pl.when(step == 0)
    def _prep():
        cbw_s[...] = cb * (2.0 * w)
        nb_s[...] = (-w) * jnp.sum(cb * cb, axis=1)[None, :]
        rev_s[...] = (jnp.int32(size) - lax.broadcasted_iota(
            jnp.int32, (1, size), 1)).astype(jnp.float32)

    rev = rev_s[...]
    loss = jnp.float32(0.0)
    for i in range(NB2):
        zb = z_ref[i]
        g = lax.dot_general(zb, cbw_s[...], (((0,), (1,)), ((), ())),
                            preferred_element_type=jnp.float32) + nb_s[...]

        rowmax = jnp.max(g, axis=1)
        e = jnp.exp(g - rowmax[:, None])
        denom = jnp.sum(e, axis=1)
        sumeg = jnp.sum(e * g, axis=1)
        kld = jnp.sum(sumeg / denom - rowmax - jnp.log(denom))

        vmax = jnp.max(jnp.where(g == rowmax[:, None], rev, 0.0), axis=1)
        onehot = (rev == vmax[:, None]).astype(jnp.float32)
        zq_ref[i] = lax.dot_general(cb, onehot, (((0,), (1,)), ((), ())),
                                    preferred_element_type=jnp.float32)

        loss += kld + w * jnp.sum(zb * zb) - jnp.sum(rowmax)

    @pl.when(step == 0)
    def _init():
        loss_ref[0, 0] = 0.0

    loss_ref[0, 0] += loss * inv_bs


def kernel(z, codebook, var_q, var_init):
    bs, dim_z, d1, d2 = z.shape
    size, _ = codebook.shape
    npix = d1 * d2
    z3 = z.reshape(bs, dim_z, npix)

    var_q_eff = jax.nn.sigmoid(var_q) * 2.0 * var_init
    w = (0.5 / jnp.clip(var_q_eff, 1e-10, None)).reshape(1, 1)

    body = functools.partial(_vq_body, size=size, inv_bs=1.0 / bs)
    zq3, loss = pl.pallas_call(
        body,
        grid=(bs // NB2,),
        in_specs=[
            pl.BlockSpec(memory_space=pltpu.SMEM),
            pl.BlockSpec((NB2, dim_z, npix), lambda s: (s, 0, 0)),
            pl.BlockSpec((size, dim_z), lambda s: (0, 0)),
        ],
        out_specs=[
            pl.BlockSpec((NB2, dim_z, npix), lambda s: (s, 0, 0)),
            pl.BlockSpec(memory_space=pltpu.SMEM),
        ],
        out_shape=[
            jax.ShapeDtypeStruct((bs, dim_z, npix), jnp.float32),
            jax.ShapeDtypeStruct((1, 1), jnp.float32),
        ],
        scratch_shapes=[
            pltpu.VMEM((size, dim_z), jnp.float32),
            pltpu.VMEM((1, size), jnp.float32),
            pltpu.VMEM((1, size), jnp.float32),
        ],
    )(w, z3, codebook)
    return zq3.reshape(bs, dim_z, d1, d2), loss[0, 0]


# speedup vs baseline: 1.4823x; 1.0327x over previous
import functools

import jax
import jax.numpy as jnp
from jax import lax
from jax.experimental import pallas as pl
from jax.experimental.pallas import tpu as pltpu

NB2 = 4


def _vq_body(w_ref, z_ref, cb_ref, zq_ref, loss_ref, cbw_s, nb_s, rev_s, *,
             size, inv_bs):
    step = pl.program_id(0)
    w = w_ref[0, 0]
    cb = cb_ref[...]

    @pl.when(step == 0)
    def _prep():
        cbw_s[...] = cb * (2.0 * w)
        nb_s[...] = (-w) * jnp.sum(cb * cb, axis=1)[None, :]
        rev_s[...] = (jnp.int32(size) - lax.broadcasted_iota(
            jnp.int32, (1, size), 1)).astype(jnp.float32)

    rev = rev_s[...]
    loss = jnp.float32(0.0)
    for i in range(NB2):
        zb = z_ref[i]
        g = lax.dot_general(zb, cbw_s[...], (((0,), (1,)), ((), ())),
                            preferred_element_type=jnp.float32) + nb_s[...]

        rowmax = jnp.max(g, axis=1)
        e = jnp.exp(g - rowmax[:, None])
        denom = jnp.sum(e, axis=1)
        sumeg = jnp.sum(e * g, axis=1)
        kld = jnp.sum(sumeg / denom - rowmax - jnp.log(denom))

        vmax = jnp.max(jnp.where(g == rowmax[:, None], rev, 0.0), axis=1)
        onehot = (rev == vmax[:, None]).astype(jnp.float32)
        zq_ref[i] = lax.dot_general(cb, onehot, (((0,), (1,)), ((), ())),
                                    preferred_element_type=jnp.float32)

        loss += kld + w * jnp.sum(zb * zb) - jnp.sum(rowmax)

    @pl.when(step == 0)
    def _init():
        loss_ref[0, 0] = 0.0

    loss_ref[0, 0] += loss * inv_bs


def kernel(z, codebook, var_q, var_init):
    bs, dim_z, d1, d2 = z.shape
    size, _ = codebook.shape
    npix = d1 * d2
    z3 = z.reshape(bs, dim_z, npix)

    var_q_eff = jax.nn.sigmoid(var_q) * 2.0 * var_init
    w = (0.5 / jnp.clip(var_q_eff, 1e-10, None)).reshape(1, 1)

    body = functools.partial(_vq_body, size=size, inv_bs=1.0 / bs)
    zq3, loss = pl.pallas_call(
        body,
        grid=(bs // NB2,),
        in_specs=[
            pl.BlockSpec(memory_space=pltpu.SMEM),
            pl.BlockSpec((NB2, dim_z, npix), lambda s: (s, 0, 0)),
            pl.BlockSpec((size, dim_z), lambda s: (0, 0)),
        ],
        out_specs=[
            pl.BlockSpec((NB2, dim_z, npix), lambda s: (s, 0, 0)),
            pl.BlockSpec(memory_space=pltpu.SMEM),
        ],
        out_shape=[
            jax.ShapeDtypeStruct((bs, dim_z, npix), jnp.float32),
            jax.ShapeDtypeStruct((1, 1), jnp.float32),
        ],
        scratch_shapes=[
            pltpu.VMEM((size, dim_z), jnp.float32),
            pltpu.VMEM((1, size), jnp.float32),
            pltpu.VMEM((1, size), jnp.float32),
        ],
    )(w, z3, codebook)
    return zq3.reshape(bs, dim_z, d1, d2), loss[0, 0]
